# baseline (device time: 122467 ns/iter reference)
import jax
import jax.numpy as jnp
from jax import lax
from jax.experimental import pallas as pl
from jax.experimental.pallas import tpu as pltpu

N_DEV = 4


def _ring_all_reduce(partial):
    t, d = partial.shape

    def body(p_ref, out_ref, comm_ref, send_sems, recv_sems):
        my_pos = lax.axis_index("i")
        left = (my_pos - 1) % N_DEV
        right = (my_pos + 1) % N_DEV

        barrier_sem = pltpu.get_barrier_semaphore()
        for nbr in [left, right]:
            pl.semaphore_signal(
                barrier_sem, inc=1,
                device_id=(nbr,), device_id_type=pl.DeviceIdType.MESH,
            )
        pl.semaphore_wait(barrier_sem, 2)

        out_ref[:, :] = p_ref[:, :].astype(jnp.float32)
        comm_ref[0, :, :] = p_ref[:, :]

        for h in range(N_DEV - 1):
            send_slot = h % 2
            recv_slot = (h + 1) % 2
            rdma = pltpu.make_async_remote_copy(
                src_ref=comm_ref.at[send_slot],
                dst_ref=comm_ref.at[recv_slot],
                send_sem=send_sems.at[send_slot],
                recv_sem=recv_sems.at[recv_slot],
                device_id=(right,),
                device_id_type=pl.DeviceIdType.MESH,
            )
            rdma.start()
            rdma.wait()
            out_ref[:, :] = out_ref[:, :] + comm_ref[recv_slot, :, :].astype(
                jnp.float32
            )

    return pl.pallas_call(
        body,
        out_shape=jax.ShapeDtypeStruct((t, d), jnp.float32),
        in_specs=[pl.BlockSpec(memory_space=pltpu.VMEM)],
        out_specs=pl.BlockSpec(memory_space=pltpu.VMEM),
        scratch_shapes=[
            pltpu.VMEM((2, t, d), partial.dtype),
            pltpu.SemaphoreType.DMA((2,)),
            pltpu.SemaphoreType.DMA((2,)),
        ],
        compiler_params=pltpu.CompilerParams(collective_id=0),
    )(partial)


def kernel(ids, E):
    v_per = E.shape[0]
    my_pos = lax.axis_index("i")
    local = ids - my_pos * v_per
    in_range = (local >= 0) & (local < v_per)
    safe = jnp.where(in_range, local, 0)
    partial = jnp.where(in_range[:, None], E[safe], 0.0).astype(jnp.bfloat16)
    return _ring_all_reduce(partial)


# device time: 60743 ns/iter; 2.0162x vs baseline; 2.0162x over previous
import jax
import jax.numpy as jnp
from jax import lax
from jax.experimental import pallas as pl
from jax.experimental.pallas import tpu as pltpu

N_DEV = 4
T = 1024
D = 1024
V_PER = 8192


def kernel(ids, E):
    def body(ids_smem, ids_vmem, e_hbm, out_ref, g_ref, acc_ref, slot_ref,
             send_sems, recv_sems, gather_sem):
        my_pos = lax.axis_index("i")
        p_a = my_pos ^ 1
        p_b = my_pos ^ 3
        b0 = my_pos & 1
        b1 = (my_pos >> 1) & 1
        lo = my_pos * V_PER

        def issue(t, _):
            idx = ids_smem[t] - lo
            safe = jnp.clip(idx, 0, V_PER - 1)
            pltpu.make_async_copy(
                e_hbm.at[pl.ds(safe, 1), :],
                g_ref.at[pl.ds(t, 1), :],
                gather_sem,
            ).start()
            return _

        lax.fori_loop(0, T, issue, None)

        barrier_sem = pltpu.get_barrier_semaphore()
        for nbr in [p_a, p_b]:
            pl.semaphore_signal(
                barrier_sem, inc=1,
                device_id=(nbr,), device_id_type=pl.DeviceIdType.MESH,
            )
        pl.semaphore_wait(barrier_sem, 2)

        pltpu.make_async_copy(
            e_hbm.at[pl.ds(0, T), :], g_ref.at[...], gather_sem
        ).wait()

        m = (ids_vmem[:, :] >= lo) & (ids_vmem[:, :] < lo + V_PER)
        acc_ref[:, :] = jnp.where(
            m, g_ref[:, :].astype(jnp.bfloat16), jnp.bfloat16(0)
        )

        halves = [
            (0, b0 ^ b1, b1, p_a, p_b),
            (T // 2, b1, b0, p_b, p_a),
        ]

        def exchange(items):
            rdmas = []
            for s, partner, send_off, size in items:
                r = pltpu.make_async_remote_copy(
                    src_ref=acc_ref.at[pl.ds(send_off, size), :],
                    dst_ref=slot_ref.at[s, pl.ds(0, size), :],
                    send_sem=send_sems.at[s],
                    recv_sem=recv_sems.at[s],
                    device_id=(partner,),
                    device_id_type=pl.DeviceIdType.MESH,
                )
                r.start()
                rdmas.append(r)
            for r in rdmas:
                r.wait()

        items = []
        for h, (base, k1, k2, p1, p2) in enumerate(halves):
            items.append((0 * 2 + h, p1, base + (1 - k1) * 256, 256))
        exchange(items)
        for h, (base, k1, k2, p1, p2) in enumerate(halves):
            off = base + k1 * 256
            acc_ref[pl.ds(off, 256), :] = (
                acc_ref[pl.ds(off, 256), :] + slot_ref[0 * 2 + h, :, :]
            )

        items = []
        for h, (base, k1, k2, p1, p2) in enumerate(halves):
            items.append(
                (1 * 2 + h, p2, base + k1 * 256 + (1 - k2) * 128, 128)
            )
        exchange(items)
        for h, (base, k1, k2, p1, p2) in enumerate(halves):
            own = base + k1 * 256 + k2 * 128
            acc_ref[pl.ds(own, 128), :] = (
                acc_ref[pl.ds(own, 128), :]
                + slot_ref[1 * 2 + h, pl.ds(0, 128), :]
            )

        items = []
        for h, (base, k1, k2, p1, p2) in enumerate(halves):
            items.append((2 * 2 + h, p2, base + k1 * 256 + k2 * 128, 128))
        exchange(items)
        for h, (base, k1, k2, p1, p2) in enumerate(halves):
            other = base + k1 * 256 + (1 - k2) * 128
            acc_ref[pl.ds(other, 128), :] = slot_ref[2 * 2 + h, pl.ds(0, 128), :]

        items = []
        for h, (base, k1, k2, p1, p2) in enumerate(halves):
            items.append((3 * 2 + h, p1, base + k1 * 256, 256))
        exchange(items)
        for h, (base, k1, k2, p1, p2) in enumerate(halves):
            off = base + (1 - k1) * 256
            acc_ref[pl.ds(off, 256), :] = slot_ref[3 * 2 + h, :, :]

        out_ref[:, :] = acc_ref[:, :].astype(jnp.float32)

    return pl.pallas_call(
        body,
        out_shape=jax.ShapeDtypeStruct((T, D), jnp.float32),
        in_specs=[
            pl.BlockSpec(memory_space=pltpu.SMEM),
            pl.BlockSpec(memory_space=pltpu.VMEM),
            pl.BlockSpec(memory_space=pltpu.MemorySpace.HBM),
        ],
        out_specs=pl.BlockSpec(memory_space=pltpu.VMEM),
        scratch_shapes=[
            pltpu.VMEM((T, D), jnp.float32),
            pltpu.VMEM((T, D), jnp.bfloat16),
            pltpu.VMEM((8, 256, D), jnp.bfloat16),
            pltpu.SemaphoreType.DMA((8,)),
            pltpu.SemaphoreType.DMA((8,)),
            pltpu.SemaphoreType.DMA,
        ],
        compiler_params=pltpu.CompilerParams(collective_id=0),
    )(ids, ids.reshape(T, 1), E)


# device time: 34221 ns/iter; 3.5787x vs baseline; 1.7750x over previous
import os

import jax
import jax.numpy as jnp
from jax import lax
from jax.experimental import pallas as pl
from jax.experimental.pallas import tpu as pltpu

N_DEV = 4
T = 1024
D = 1024
V_PER = 8192
Q = T // 4

_SKIP_AR = os.environ.get("K_SKIP_AR") == "1"
_SKIP_GATHER = os.environ.get("K_SKIP_GATHER") == "1"


def kernel(ids, E):
    my_pos = lax.axis_index("i")
    b0 = my_pos & 1
    b1 = (my_pos >> 1) & 1
    k1a = b0 ^ b1
    k1b = b1

    local = ids - my_pos * V_PER
    ok = (local >= 0) & (local < V_PER)
    tok = jnp.arange(T, dtype=jnp.int32)
    sa = (1 - k1a) * Q
    sb = 2 * Q + (1 - k1b) * Q
    in_send = ((tok >= sa) & (tok < sa + Q)) | ((tok >= sb) & (tok < sb + Q))
    key = jnp.where(ok, jnp.where(in_send, 0, 1), 2)
    loc = jnp.clip(local, 0, V_PER - 1)
    packed = lax.sort((key << 23) | (loc << 10) | tok)

    def body(pk_smem, ids_vmem, e_hbm, out_ref,
             g_ref, slot_ref, send_sems, recv_sems, gather_sems):
        my = lax.axis_index("i")
        p_a = my ^ 1
        p_b = my ^ 3
        vb0 = my & 1
        vb1 = (my >> 1) & 1
        vlo = my * V_PER

        def count_below(thresh):
            def step(_, lohi):
                sl, sh = lohi
                mid = (sl + sh) // 2
                big = pk_smem[mid] >= thresh
                return (jnp.where(big, sl, mid + 1),
                        jnp.where(big, mid, sh))
            sl, _ = lax.fori_loop(
                0, 10, step, (jnp.int32(0), jnp.int32(T))
            )
            return sl

        ns = count_below(jnp.int32(1 << 23))
        nh = count_below(jnp.int32(2 << 23))

        def make_issue(sem_i):
            def issue(i, _):
                v = pk_smem[i]
                pltpu.make_async_copy(
                    e_hbm.at[pl.ds((v >> 10) & (V_PER - 1), 1), :],
                    g_ref.at[pl.ds(v & (T - 1), 1), :],
                    gather_sems.at[sem_i],
                ).start()
                return _
            return issue

        if not _SKIP_GATHER:
            lax.fori_loop(0, ns, make_issue(0), None)

        barrier_sem = pltpu.get_barrier_semaphore()
        for nbr in [p_a, p_b]:
            pl.semaphore_signal(
                barrier_sem, inc=1,
                device_id=(nbr,), device_id_type=pl.DeviceIdType.MESH,
            )
        pl.semaphore_wait(barrier_sem, 2)

        def wait_rows(sem_i, count):
            def waitone(i, _):
                pltpu.make_async_copy(
                    e_hbm.at[pl.ds(0, 1), :], g_ref.at[pl.ds(0, 1), :],
                    gather_sems.at[sem_i],
                ).wait()
                return _
            lax.fori_loop(0, count, waitone, None)

        def mask_block(off, co=0, cw=D):
            m = (ids_vmem[pl.ds(off, Q), :] >= vlo) & (
                ids_vmem[pl.ds(off, Q), :] < vlo + V_PER
            )
            out_ref[pl.ds(off, Q), pl.ds(co, cw)] = jnp.where(
                m, g_ref[pl.ds(off, Q), pl.ds(co, cw)].astype(jnp.bfloat16),
                jnp.bfloat16(0),
            )

        halves = [
            (0, vb0 ^ vb1, vb1, p_a, p_b),
            (T // 2, vb1, vb0, p_b, p_a),
        ]
        send_offs = [base + (1 - k1) * Q for base, k1, _, _, _ in halves]
        keep_offs = [base + k1 * Q for base, k1, _, _, _ in halves]

        if not _SKIP_GATHER:
            wait_rows(0, ns)

        if _SKIP_AR:
            if not _SKIP_GATHER:
                lax.fori_loop(ns, nh, make_issue(1), None)
                wait_rows(1, nh - ns)
            for off in send_offs + keep_offs:
                mask_block(off)
            return

        NC = 4
        CW = D // NC

        own_offs = [
            base + k1 * Q + k2 * 128 for base, k1, k2, _, _ in halves
        ]
        other_offs = [
            base + k1 * Q + (1 - k2) * 128 for base, k1, k2, _, _ in halves
        ]
        phase_spec = [
            (Q, send_offs, [halves[h][3] for h in (0, 1)]),
            (128, [base + k1 * Q + (1 - k2) * 128
                   for base, k1, k2, _, _ in halves],
             [halves[h][4] for h in (0, 1)]),
            (128, own_offs, [halves[h][4] for h in (0, 1)]),
            (Q, keep_offs, [halves[h][3] for h in (0, 1)]),
        ]

        def start_phase(p, c):
            rows, offs, partners = phase_spec[p]
            rdmas = []
            for h in (0, 1):
                s = (p * 2 + h) * NC + c
                r = pltpu.make_async_remote_copy(
                    src_ref=out_ref.at[pl.ds(offs[h], rows),
                                       pl.ds(c * CW, CW)],
                    dst_ref=slot_ref.at[s, pl.ds(0, rows), :],
                    send_sem=send_sems.at[s],
                    recv_sem=recv_sems.at[s],
                    device_id=(partners[h],),
                    device_id_type=pl.DeviceIdType.MESH,
                )
                r.start()
                rdmas.append(r)
            return rdmas

        def finish_phase(p, c, rdmas):
            for r in rdmas:
                r.wait()
            dest = [keep_offs, own_offs, other_offs, send_offs][p]
            rows = phase_spec[p][0]
            for h in (0, 1):
                s = (p * 2 + h) * NC + c
                dst = out_ref.at[pl.ds(dest[h], rows), pl.ds(c * CW, CW)]
                if p <= 1:
                    dst[...] = dst[...] + slot_ref[s, pl.ds(0, rows), :]
                else:
                    dst[...] = slot_ref[s, pl.ds(0, rows), :]

        inflight = [None] * NC
        for c in range(NC):
            for off in send_offs:
                mask_block(off, c * CW, CW)
            inflight[c] = start_phase(0, c)

        if not _SKIP_GATHER:
            lax.fori_loop(ns, nh, make_issue(1), None)
            wait_rows(1, nh - ns)
        for off in keep_offs:
            mask_block(off)

        for p in range(4):
            for c in range(NC):
                finish_phase(p, c, inflight[c])
                if p < 3:
                    inflight[c] = start_phase(p + 1, c)

    return pl.pallas_call(
        body,
        out_shape=jax.ShapeDtypeStruct((T, D), jnp.bfloat16),
        in_specs=[
            pl.BlockSpec(memory_space=pltpu.SMEM),
            pl.BlockSpec(memory_space=pltpu.VMEM),
            pl.BlockSpec(memory_space=pltpu.MemorySpace.HBM),
        ],
        out_specs=pl.BlockSpec(memory_space=pltpu.VMEM),
        scratch_shapes=[
            pltpu.VMEM((T, D), jnp.float32),
            pltpu.VMEM((32, Q, D // 4), jnp.bfloat16),
            pltpu.SemaphoreType.DMA((32,)),
            pltpu.SemaphoreType.DMA((32,)),
            pltpu.SemaphoreType.DMA((2,)),
        ],
        compiler_params=pltpu.CompilerParams(collective_id=0),
    )(packed, ids.reshape(T, 1), E)
